# baseline (speedup 1.0000x reference)
import functools

import numpy as np

import jax
import jax.numpy as jnp
from jax import lax
from jax.experimental import pallas as pl
from jax.experimental.pallas import tpu as pltpu
from jax.experimental.pallas import tpu_sc as plsc

_INPUT_DIM = 4
_L = 16


@functools.lru_cache(maxsize=None)
def _build_sc_call(batch, node_num, input_dim, init_dim, num_edges):
    nc, ns = 2, 16
    nw = nc * ns
    nb = batch // nw
    out_d = input_dim + init_dim
    ncols = batch * node_num
    assert ncols % (128 * nw) == 0 and (ncols // nw) % (3 * 128) == 0
    wtiles = ncols // 128 // nw
    obs_chunk = (nb // 128) * 1024
    ei_words = nb * num_edges
    obs_off = 16
    n_macro = wtiles // node_num
    ei_step = node_num * (_L // num_edges)
    n_ec = ei_words // 128

    lane = np.arange(_L)
    tabs = []
    for d in range(input_dim):
        for tr in range(node_num):
            for s in range(128 // _L):
                cc = _L * s + lane
                bl = (128 * tr + cc) // node_num
                n = (2 * tr + cc) % node_num
                q = input_dim * n + d
                tabs.append((obs_off + (q // 8) * obs_chunk
                             + (bl // 128) * 1024 + (q % 8) * 128
                             + bl % 128).astype(np.int32))
    for d in range(out_d - input_dim):
        for rr in range(node_num):
            n = (rr + lane) % node_num
            tabs.append(np.where(n == 0, d, 8 + d).astype(np.int32))
    tabs.append(((lane // num_edges) * node_num).astype(np.int32))
    tabs.append((lane % num_edges).astype(np.int32))
    tabs.append((num_edges + lane % num_edges).astype(np.int32))
    consts_np = np.concatenate(tabs)
    n_const = consts_np.shape[0]
    n_xvec = input_dim * node_num * (128 // _L)

    mesh = plsc.VectorSubcoreMesh(core_axis_name="c", subcore_axis_name="s")

    @functools.partial(
        pl.kernel,
        out_type=[
            jax.ShapeDtypeStruct((8 * ncols,), jnp.float32),
            jax.ShapeDtypeStruct((2 * batch * num_edges,), jnp.int32),
        ],
        mesh=mesh,
        compiler_params=pltpu.CompilerParams(needs_layout_passes=False),
        scratch_types=[
            pltpu.VMEM((obs_off + 2 * obs_chunk,), jnp.float32),
            pltpu.VMEM((wtiles * 1024,), jnp.float32),
            pltpu.VMEM((2 * num_edges,), jnp.int32),
            pltpu.VMEM((2 * ei_words,), jnp.int32),
            pltpu.VMEM((n_const,), jnp.int32),
            pltpu.SemaphoreType.DMA,
            pltpu.SemaphoreType.DMA,
            pltpu.SemaphoreType.DMA,
        ],
    )
    def sc_call(o0_hbm, o1_hbm, e0_hbm, e1_hbm, ei_hbm, consts_hbm,
                x_hbm, eiout_hbm,
                stage, outbuf, eist, eibuf, constv, sem_a, sem_b, sem_o):
        wid = lax.axis_index("s") * nc + lax.axis_index("c")
        base = wid * nb
        h_small = [
            pltpu.async_copy(e0_hbm, stage.at[pl.ds(0, init_dim)], sem_a),
            pltpu.async_copy(e1_hbm, stage.at[pl.ds(8, init_dim)], sem_a),
            pltpu.async_copy(ei_hbm, eist, sem_a),
            pltpu.async_copy(consts_hbm, constv, sem_a),
        ]
        h_obs = [
            pltpu.async_copy(o0_hbm.at[pl.ds(wid * obs_chunk, obs_chunk)],
                             stage.at[pl.ds(obs_off, obs_chunk)], sem_b),
            pltpu.async_copy(o1_hbm.at[pl.ds(wid * obs_chunk, obs_chunk)],
                             stage.at[pl.ds(obs_off + obs_chunk, obs_chunk)],
                             sem_b),
        ]
        for h in h_small:
            h.wait()

        ce = n_xvec + 2 * node_num
        tri = constv[pl.ds(_L * ce, _L)] + node_num * base
        b0 = plsc.load_gather(eist, [constv[pl.ds(_L * (ce + 1), _L)]]) + tri
        b1 = plsc.load_gather(eist, [constv[pl.ds(_L * (ce + 2), _L)]]) + tri
        bs = [(b0 + s * ei_step, b1 + s * ei_step)
              for s in range(128 // _L)]

        def eibody(c, carry):
            cc = c * ((128 // _L) * ei_step)
            co = c * 256
            for s in range(128 // _L):
                eibuf[pl.ds(co + _L * s, _L)] = bs[s][0] + cc
                eibuf[pl.ds(co + 128 + _L * s, _L)] = bs[s][1] + cc
            return carry
        lax.fori_loop(0, n_ec, eibody, 0, unroll=False)

        h_out = [
            pltpu.async_copy(
                eibuf,
                eiout_hbm.at[pl.ds(wid * (2 * ei_words), 2 * ei_words)],
                sem_o),
        ]
        for h in h_obs:
            h.wait()

        xbase = wid * (wtiles * 1024)
        for lo, hi in ((0, n_macro // 2), (n_macro // 2, n_macro)):
            for d in range(input_dim):
                kb = [constv[pl.ds(_L * (d * 24 + j), _L)] for j in range(24)]

                def xbody(mi, carry, _d=d, _kb=kb):
                    mo = mi * (node_num * 1024) + _d * 128
                    sst = stage.at[pl.ds(mi * 1024,
                                         obs_off + obs_chunk + 1024)]
                    for tr in range(node_num):
                        for s in range(128 // _L):
                            outbuf[pl.ds(mo + tr * 1024 + _L * s, _L)] = \
                                plsc.load_gather(sst, [_kb[tr * 8 + s]])
                    return carry
                lax.fori_loop(lo, hi, xbody, 0, unroll=2)

            for d in range(input_dim, out_d):
                vals = [plsc.load_gather(
                            stage,
                            [constv[pl.ds(_L * (n_xvec + (d - input_dim)
                                                * node_num + rr), _L)]])
                        for rr in range(node_num)]

                def ibody(mi, carry, _d=d, _vals=vals):
                    mo = mi * (node_num * 1024) + _d * 128
                    for tr in range(node_num):
                        for s in range(128 // _L):
                            rr = (2 * tr + _L * s) % node_num
                            outbuf[pl.ds(mo + tr * 1024 + _L * s, _L)] = \
                                _vals[rr]
                    return carry
                lax.fori_loop(lo, hi, ibody, 0, unroll=False)

            for t in range(lo * node_num, hi * node_num):
                h_out.append(pltpu.async_copy(
                    outbuf.at[pl.ds(t * 1024, 768)],
                    x_hbm.at[pl.ds(xbase + t * 1024, 768)], sem_o))

        for h in h_out:
            h.wait()

    return sc_call, consts_np


def kernel(obs, ego_init, other_init, edge_index):
    batch, obs_dim = obs.shape
    node_num = obs_dim // _INPUT_DIM
    init_dim = ego_init.shape[0]
    num_edges = edge_index.shape[1]
    sc_call, consts_np = _build_sc_call(batch, node_num, _INPUT_DIM,
                                        init_dim, num_edges)
    nrow = node_num * _INPUT_DIM
    ncols = batch * node_num
    o0 = obs[:, :8].T.reshape(8, batch // 128, 128) \
                     .transpose(1, 0, 2).reshape(-1)
    o1 = jnp.concatenate(
        [obs[:, 8:].T, jnp.zeros((16 - nrow, batch), obs.dtype)], axis=0) \
        .reshape(8, batch // 128, 128).transpose(1, 0, 2).reshape(-1)
    xphys, ei_img = sc_call(o0, o1, ego_init, other_init,
                            edge_index.reshape(-1),
                            jnp.asarray(consts_np))
    x = xphys.reshape(ncols // 128, 8, 128).transpose(1, 0, 2) \
             .reshape(8, ncols)[:_INPUT_DIM + init_dim].T
    ei = ei_img.reshape(batch * num_edges // 128, 2, 128) \
               .transpose(1, 0, 2).reshape(2, batch * num_edges)
    return x, ei


# speedup vs baseline: 1.0208x; 1.0208x over previous
import functools

import numpy as np

import jax
import jax.numpy as jnp
from jax import lax
from jax.experimental import pallas as pl
from jax.experimental.pallas import tpu as pltpu
from jax.experimental.pallas import tpu_sc as plsc

_INPUT_DIM = 4
_L = 16


@functools.lru_cache(maxsize=None)
def _build_sc_call(batch, node_num, input_dim, init_dim, num_edges):
    nc, ns = 2, 16
    nw = nc * ns
    nb = batch // nw
    out_d = input_dim + init_dim
    ncols = batch * node_num
    assert ncols % (128 * nw) == 0 and (ncols // nw) % (3 * 128) == 0
    wtiles = ncols // 128 // nw
    obs_chunk = (nb // 128) * 1024
    ei_words = nb * num_edges
    obs_off = 16
    n_macro = wtiles // node_num
    ei_step = node_num * (_L // num_edges)
    n_ec = ei_words // 128

    lane = np.arange(_L)
    tabs = []
    for d in range(input_dim):
        for tr in range(node_num):
            for s in range(128 // _L):
                cc = _L * s + lane
                bl = (128 * tr + cc) // node_num
                n = (2 * tr + cc) % node_num
                q = input_dim * n + d
                tabs.append((obs_off + (q // 8) * obs_chunk
                             + (bl // 128) * 1024 + (q % 8) * 128
                             + bl % 128).astype(np.int32))
    for d in range(out_d - input_dim):
        for rr in range(node_num):
            n = (rr + lane) % node_num
            tabs.append(np.where(n == 0, d, 8 + d).astype(np.int32))
    tabs.append(((lane // num_edges) * node_num).astype(np.int32))
    tabs.append((lane % num_edges).astype(np.int32))
    tabs.append((num_edges + lane % num_edges).astype(np.int32))
    consts_np = np.concatenate(tabs)
    n_const = consts_np.shape[0]
    n_xvec = input_dim * node_num * (128 // _L)

    mesh = plsc.VectorSubcoreMesh(core_axis_name="c", subcore_axis_name="s")

    @functools.partial(
        pl.kernel,
        out_type=[
            jax.ShapeDtypeStruct((8 * ncols,), jnp.float32),
            jax.ShapeDtypeStruct((2 * batch * num_edges,), jnp.int32),
        ],
        mesh=mesh,
        compiler_params=pltpu.CompilerParams(needs_layout_passes=False),
        scratch_types=[
            pltpu.VMEM((obs_off + 2 * obs_chunk,), jnp.float32),
            pltpu.VMEM((wtiles * 1024,), jnp.float32),
            pltpu.VMEM((2 * num_edges,), jnp.int32),
            pltpu.VMEM((2 * ei_words,), jnp.int32),
            pltpu.VMEM((n_const,), jnp.int32),
            pltpu.SemaphoreType.DMA,
            pltpu.SemaphoreType.DMA,
            pltpu.SemaphoreType.DMA,
        ],
    )
    def sc_call(o0_hbm, o1_hbm, e0_hbm, e1_hbm, ei_hbm, consts_hbm,
                x_hbm, eiout_hbm,
                stage, outbuf, eist, eibuf, constv, sem_a, sem_b, sem_o):
        wid = lax.axis_index("s") * nc + lax.axis_index("c")
        base = wid * nb
        h_small = [
            pltpu.async_copy(e0_hbm, stage.at[pl.ds(0, init_dim)], sem_a),
            pltpu.async_copy(e1_hbm, stage.at[pl.ds(8, init_dim)], sem_a),
            pltpu.async_copy(ei_hbm, eist, sem_a),
            pltpu.async_copy(consts_hbm, constv, sem_a),
        ]
        h_obs = [
            pltpu.async_copy(o0_hbm.at[pl.ds(wid * obs_chunk, obs_chunk)],
                             stage.at[pl.ds(obs_off, obs_chunk)], sem_b),
            pltpu.async_copy(o1_hbm.at[pl.ds(wid * obs_chunk, obs_chunk)],
                             stage.at[pl.ds(obs_off + obs_chunk, obs_chunk)],
                             sem_b),
        ]
        for h in h_small:
            h.wait()

        ce = n_xvec + 2 * node_num
        tri = constv[pl.ds(_L * ce, _L)] + node_num * base
        b0 = plsc.load_gather(eist, [constv[pl.ds(_L * (ce + 1), _L)]]) + tri
        b1 = plsc.load_gather(eist, [constv[pl.ds(_L * (ce + 2), _L)]]) + tri
        bs = [(b0 + s * ei_step, b1 + s * ei_step)
              for s in range(128 // _L)]

        def eibody(c, carry):
            cc = c * ((128 // _L) * ei_step)
            co = c * 256
            for s in range(128 // _L):
                eibuf[pl.ds(co + _L * s, _L)] = bs[s][0] + cc
                eibuf[pl.ds(co + 128 + _L * s, _L)] = bs[s][1] + cc
            return carry
        lax.fori_loop(0, n_ec, eibody, 0, unroll=False)

        h_out = [
            pltpu.async_copy(
                eibuf,
                eiout_hbm.at[pl.ds(wid * (2 * ei_words), 2 * ei_words)],
                sem_o),
        ]
        for h in h_obs:
            h.wait()

        xbase = wid * (wtiles * 1024)
        for lo, hi in ((0, n_macro // 2), (n_macro // 2, n_macro)):
            for d in range(input_dim):
                kb = [constv[pl.ds(_L * (d * 24 + j), _L)] for j in range(24)]

                def xbody(mi, carry, _d=d, _kb=kb):
                    mo = mi * (node_num * 1024) + _d * 128
                    sst = stage.at[pl.ds(mi * 1024,
                                         obs_off + obs_chunk + 1024)]
                    for tr in range(node_num):
                        for s in range(128 // _L):
                            outbuf[pl.ds(mo + tr * 1024 + _L * s, _L)] = \
                                plsc.load_gather(sst, [_kb[tr * 8 + s]])
                    return carry
                lax.fori_loop(lo, hi, xbody, 0, unroll=False)

            for d in range(input_dim, out_d):
                vals = [plsc.load_gather(
                            stage,
                            [constv[pl.ds(_L * (n_xvec + (d - input_dim)
                                                * node_num + rr), _L)]])
                        for rr in range(node_num)]

                def ibody(mi, carry, _d=d, _vals=vals):
                    mo = mi * (node_num * 1024) + _d * 128
                    for tr in range(node_num):
                        for s in range(128 // _L):
                            rr = (2 * tr + _L * s) % node_num
                            outbuf[pl.ds(mo + tr * 1024 + _L * s, _L)] = \
                                _vals[rr]
                    return carry
                lax.fori_loop(lo, hi, ibody, 0, unroll=False)

            for t in range(lo * node_num, hi * node_num):
                h_out.append(pltpu.async_copy(
                    outbuf.at[pl.ds(t * 1024, 768)],
                    x_hbm.at[pl.ds(xbase + t * 1024, 768)], sem_o))

        for h in h_out:
            h.wait()

    return sc_call, consts_np


def kernel(obs, ego_init, other_init, edge_index):
    batch, obs_dim = obs.shape
    node_num = obs_dim // _INPUT_DIM
    init_dim = ego_init.shape[0]
    num_edges = edge_index.shape[1]
    sc_call, consts_np = _build_sc_call(batch, node_num, _INPUT_DIM,
                                        init_dim, num_edges)
    nrow = node_num * _INPUT_DIM
    ncols = batch * node_num
    o0 = obs[:, :8].T.reshape(8, batch // 128, 128) \
                     .transpose(1, 0, 2).reshape(-1)
    o1 = jnp.concatenate(
        [obs[:, 8:].T, jnp.zeros((16 - nrow, batch), obs.dtype)], axis=0) \
        .reshape(8, batch // 128, 128).transpose(1, 0, 2).reshape(-1)
    xphys, ei_img = sc_call(o0, o1, ego_init, other_init,
                            edge_index.reshape(-1),
                            jnp.asarray(consts_np))
    x = xphys.reshape(ncols // 128, 8, 128).transpose(1, 0, 2) \
             .reshape(8, ncols)[:_INPUT_DIM + init_dim].T
    ei = ei_img.reshape(batch * num_edges // 128, 2, 128) \
               .transpose(1, 0, 2).reshape(2, batch * num_edges)
    return x, ei
